# baseline (device time: 70810 ns/iter reference)
import jax
import jax.numpy as jnp
from jax import lax
from jax.experimental import pallas as pl
from jax.experimental.pallas import tpu as pltpu

N_DEV = 4
B, SQ, D = 2, 256, 768
HQ, HKV, DH, SKV = 32, 8, 64, 512
HQ_LOC = HQ // N_DEV
HKV_LOC = HKV // N_DEV
GQ = HQ // HKV
M = B * SQ


def kernel(x, Wq, Wo, K_ext, V_ext):
    my = lax.axis_index("i")
    K_loc = lax.dynamic_slice_in_dim(K_ext, my * HKV_LOC, HKV_LOC, axis=2)
    V_loc = lax.dynamic_slice_in_dim(V_ext, my * HKV_LOC, HKV_LOC, axis=2)
    kt = jnp.transpose(K_loc, (0, 2, 3, 1)).astype(jnp.bfloat16)
    vl = jnp.transpose(V_loc, (0, 2, 1, 3)).astype(jnp.bfloat16)
    x2 = x.reshape(M, D).astype(jnp.bfloat16)
    wq = Wq.astype(jnp.bfloat16)
    wo = Wo.astype(jnp.bfloat16)

    def body(x_ref, wq_ref, kt_ref, v_ref, wo_ref, out_ref,
             o_ref, comm_ref, send_sems, recv_sems):
        my_pos = lax.axis_index("i")
        left = (my_pos - 1) % N_DEV
        right = (my_pos + 1) % N_DEV

        barrier_sem = pltpu.get_barrier_semaphore()
        for nbr in (left, right):
            pl.semaphore_signal(barrier_sem, inc=1, device_id=(nbr,),
                                device_id_type=pl.DeviceIdType.MESH)

        q = jnp.dot(x_ref[...], wq_ref[...],
                    preferred_element_type=jnp.float32)
        q = q.astype(jnp.bfloat16)

        for b in range(B):
            for hl in range(HQ_LOC):
                kv = hl // GQ
                qh = q[b * SQ:(b + 1) * SQ, hl * DH:(hl + 1) * DH]
                s = jnp.dot(qh, kt_ref[b, kv],
                            preferred_element_type=jnp.float32) * 0.125
                m = jnp.max(s, axis=1, keepdims=True)
                p = jnp.exp(s - m)
                l = jnp.sum(p, axis=1, keepdims=True)
                o = jnp.dot(p.astype(jnp.bfloat16), v_ref[b, kv],
                            preferred_element_type=jnp.float32)
                o_ref[b * SQ:(b + 1) * SQ, hl * DH:(hl + 1) * DH] = \
                    (o / l).astype(jnp.bfloat16)

        partial = jnp.dot(o_ref[...], wo_ref[...],
                          preferred_element_type=jnp.float32)
        comm_ref[0, :, :] = partial
        out_ref[...] = partial

        pl.semaphore_wait(barrier_sem, 2)

        for h in range(N_DEV - 1):
            rdma = pltpu.make_async_remote_copy(
                src_ref=comm_ref.at[h],
                dst_ref=comm_ref.at[h + 1],
                send_sem=send_sems.at[h],
                recv_sem=recv_sems.at[h],
                device_id=(right,),
                device_id_type=pl.DeviceIdType.MESH,
            )
            rdma.start()
            rdma.wait()
            out_ref[...] = out_ref[...] + comm_ref[h + 1, :, :]

    out = pl.pallas_call(
        body,
        out_shape=jax.ShapeDtypeStruct((M, D), jnp.float32),
        in_specs=[pl.BlockSpec(memory_space=pltpu.VMEM)] * 5,
        out_specs=pl.BlockSpec(memory_space=pltpu.VMEM),
        scratch_shapes=[
            pltpu.VMEM((M, HQ_LOC * DH), jnp.bfloat16),
            pltpu.VMEM((N_DEV, M, D), jnp.float32),
            pltpu.SemaphoreType.DMA((N_DEV - 1,)),
            pltpu.SemaphoreType.DMA((N_DEV - 1,)),
        ],
        compiler_params=pltpu.CompilerParams(collective_id=0),
    )(x2, wq, kt, vl, wo)
    return out.reshape(B, SQ, D)


# device time: 25716 ns/iter; 2.7535x vs baseline; 2.7535x over previous
import jax
import jax.numpy as jnp
from jax import lax
from jax.experimental import pallas as pl
from jax.experimental.pallas import tpu as pltpu

N_DEV = 4
B, SQ, D = 2, 256, 768
HQ, HKV, DH, SKV = 32, 8, 64, 512
HQ_LOC = HQ // N_DEV
HKV_LOC = HKV // N_DEV
GQ = HQ // HKV
M = B * SQ
MQ = M // N_DEV


def kernel(x, Wq, Wo, K_ext, V_ext):
    my = lax.axis_index("i")
    K_loc = lax.dynamic_slice_in_dim(K_ext, my * HKV_LOC, HKV_LOC, axis=2)
    V_loc = lax.dynamic_slice_in_dim(V_ext, my * HKV_LOC, HKV_LOC, axis=2)
    kt = jnp.transpose(K_loc, (0, 2, 3, 1)).astype(jnp.bfloat16)
    vl = jnp.transpose(V_loc, (0, 2, 1, 3)).astype(jnp.bfloat16)
    x2 = x.reshape(M, D).astype(jnp.bfloat16)
    wq = Wq.astype(jnp.bfloat16)
    wo = Wo.astype(jnp.bfloat16)

    def body(x_ref, wq_ref, kt_ref, v_ref, wo_ref, out_ref,
             o_ref, pbf_ref, rs_buf, agq_ref, ag_buf,
             rs_send, rs_recv, ag_send, ag_recv):
        my_pos = lax.axis_index("i")

        barrier_sem = pltpu.get_barrier_semaphore()
        for d in range(1, N_DEV):
            pl.semaphore_signal(barrier_sem, inc=1,
                                device_id=((my_pos + d) % N_DEV,),
                                device_id_type=pl.DeviceIdType.MESH)

        q = jnp.dot(x_ref[...], wq_ref[...],
                    preferred_element_type=jnp.float32)
        q = q.astype(jnp.bfloat16)

        for b in range(B):
            for hl in range(HQ_LOC):
                kv = hl // GQ
                qh = q[b * SQ:(b + 1) * SQ, hl * DH:(hl + 1) * DH]
                s = jnp.dot(qh, kt_ref[b, kv],
                            preferred_element_type=jnp.float32) * 0.125
                m = jnp.max(s, axis=1, keepdims=True)
                p = jnp.exp(s - m)
                l = jnp.sum(p, axis=1, keepdims=True)
                o = jnp.dot(p.astype(jnp.bfloat16), v_ref[b, kv],
                            preferred_element_type=jnp.float32)
                o_ref[b * SQ:(b + 1) * SQ, hl * DH:(hl + 1) * DH] = \
                    (o / l).astype(jnp.bfloat16)

        partial = jnp.dot(o_ref[...], wo_ref[...],
                          preferred_element_type=jnp.float32)
        pbf_ref[...] = partial.astype(jnp.bfloat16)

        pl.semaphore_wait(barrier_sem, N_DEV - 1)

        rs_rdmas = []
        for d in range(1, N_DEV):
            peer = (my_pos + d) % N_DEV
            rdma = pltpu.make_async_remote_copy(
                src_ref=pbf_ref.at[pl.ds(peer * MQ, MQ), :],
                dst_ref=rs_buf.at[d - 1],
                send_sem=rs_send.at[d - 1],
                recv_sem=rs_recv.at[d - 1],
                device_id=(peer,),
                device_id_type=pl.DeviceIdType.MESH,
            )
            rdma.start()
            rs_rdmas.append(rdma)

        acc = pbf_ref[pl.ds(my_pos * MQ, MQ), :].astype(jnp.float32)
        for d in range(1, N_DEV):
            recv = pltpu.make_async_remote_copy(
                src_ref=rs_buf.at[d - 1],
                dst_ref=rs_buf.at[d - 1],
                send_sem=rs_send.at[d - 1],
                recv_sem=rs_recv.at[d - 1],
                device_id=(my_pos,),
                device_id_type=pl.DeviceIdType.MESH,
            )
            recv.wait_recv()
            acc = acc + rs_buf[d - 1].astype(jnp.float32)

        agq_ref[...] = acc.astype(jnp.bfloat16)
        ag_rdmas = []
        for d in range(1, N_DEV):
            peer = (my_pos + d) % N_DEV
            rdma = pltpu.make_async_remote_copy(
                src_ref=agq_ref,
                dst_ref=ag_buf.at[d - 1],
                send_sem=ag_send.at[d - 1],
                recv_sem=ag_recv.at[d - 1],
                device_id=(peer,),
                device_id_type=pl.DeviceIdType.MESH,
            )
            rdma.start()
            ag_rdmas.append(rdma)

        out_ref[pl.ds(my_pos * MQ, MQ), :] = acc

        for d in range(1, N_DEV):
            src_pos = (my_pos + N_DEV - d) % N_DEV
            recv = pltpu.make_async_remote_copy(
                src_ref=ag_buf.at[d - 1],
                dst_ref=ag_buf.at[d - 1],
                send_sem=ag_send.at[d - 1],
                recv_sem=ag_recv.at[d - 1],
                device_id=(my_pos,),
                device_id_type=pl.DeviceIdType.MESH,
            )
            recv.wait_recv()
            out_ref[pl.ds(src_pos * MQ, MQ), :] = \
                ag_buf[d - 1].astype(jnp.float32)

        for rdma in rs_rdmas + ag_rdmas:
            rdma.wait_send()

    out = pl.pallas_call(
        body,
        out_shape=jax.ShapeDtypeStruct((M, D), jnp.float32),
        in_specs=[pl.BlockSpec(memory_space=pltpu.VMEM)] * 5,
        out_specs=pl.BlockSpec(memory_space=pltpu.VMEM),
        scratch_shapes=[
            pltpu.VMEM((M, HQ_LOC * DH), jnp.bfloat16),
            pltpu.VMEM((M, D), jnp.bfloat16),
            pltpu.VMEM((N_DEV - 1, MQ, D), jnp.bfloat16),
            pltpu.VMEM((MQ, D), jnp.bfloat16),
            pltpu.VMEM((N_DEV - 1, MQ, D), jnp.bfloat16),
            pltpu.SemaphoreType.DMA((N_DEV - 1,)),
            pltpu.SemaphoreType.DMA((N_DEV - 1,)),
            pltpu.SemaphoreType.DMA((N_DEV - 1,)),
            pltpu.SemaphoreType.DMA((N_DEV - 1,)),
        ],
        compiler_params=pltpu.CompilerParams(collective_id=0),
    )(x2, wq, kt, vl, wo)
    return out.reshape(B, SQ, D)
